# baseline (speedup 1.0000x reference)
import jax
import jax.numpy as jnp
from jax.experimental import pallas as pl
from jax.experimental.pallas import tpu as pltpu

_TN = 256
_LC = 128


def _vq_block(x_ref, xsq_ref, cbt_ref, cb_ref,
              c2f_ref, oh_ref, xhat_ref, idx_ref):
    G = cb_ref.shape[0]
    CB = cb_ref.shape[1]
    TN = xsq_ref.shape[0]
    iota_c = jax.lax.broadcasted_iota(jnp.int32, (TN, _LC), 1)
    iota_f = jax.lax.broadcasted_iota(jnp.int32, (TN, CB), 1)
    for g in range(G):
        base = g * CB
        prod = jax.lax.dot_general(
            x_ref[:, g, :], cbt_ref[g], (((1,), (0,)), ((), ())),
            preferred_element_type=jnp.float32)
        x2g = xsq_ref[:, g:g + 1]
        rv = (x2g + c2f_ref[0:1, base:base + _LC]) + prod[:, 0:_LC]
        ri = iota_c
        for c in range(1, CB // _LC):
            lo = c * _LC
            d = (x2g + c2f_ref[0:1, base + lo:base + lo + _LC]) + prod[:, lo:lo + _LC]
            upd = d < rv
            ri = jnp.where(upd, iota_c + lo, ri)
            rv = jnp.where(upd, d, rv)
        m = jnp.min(rv, axis=1, keepdims=True)
        cand = jnp.where(rv == m, ri, CB)
        idx = jnp.min(cand, axis=1, keepdims=True)
        oh = (iota_f == idx).astype(jnp.float32)
        oh_ref[:, g, :] = oh
        xhat_ref[:, g, :] = jnp.dot(
            oh, cb_ref[g], preferred_element_type=jnp.float32)
        idx_ref[:, g:g + 1] = idx


def kernel(x, code_book):
    n, G, dim = x.shape
    CB = code_book.shape[1]
    xsq = jnp.sum(x * x, axis=-1)
    c2f = jnp.sum(code_book * code_book, axis=-1).reshape(1, G * CB)
    cbt = jnp.transpose(-2.0 * code_book, (0, 2, 1))
    one_hot, x_hat, index = pl.pallas_call(
        _vq_block,
        grid=(n // _TN,),
        in_specs=[
            pl.BlockSpec((_TN, G, dim), lambda i: (i, 0, 0)),
            pl.BlockSpec((_TN, G), lambda i: (i, 0)),
            pl.BlockSpec((G, dim, CB), lambda i: (0, 0, 0)),
            pl.BlockSpec((G, CB, dim), lambda i: (0, 0, 0)),
            pl.BlockSpec((1, G * CB), lambda i: (0, 0)),
        ],
        out_specs=[
            pl.BlockSpec((_TN, G, CB), lambda i: (i, 0, 0)),
            pl.BlockSpec((_TN, G, dim), lambda i: (i, 0, 0)),
            pl.BlockSpec((_TN, G), lambda i: (i, 0)),
        ],
        out_shape=[
            jax.ShapeDtypeStruct((n, G, CB), jnp.float32),
            jax.ShapeDtypeStruct((n, G, dim), jnp.float32),
            jax.ShapeDtypeStruct((n, G), jnp.int32),
        ],
        compiler_params=pltpu.CompilerParams(
            dimension_semantics=("parallel",)),
    )(x, xsq, cbt, code_book, c2f)
    return (x_hat, one_hot, index[..., None])


# speedup vs baseline: 1.0440x; 1.0440x over previous
import jax
import jax.numpy as jnp
from jax.experimental import pallas as pl
from jax.experimental.pallas import tpu as pltpu

_TN = 256
_LC = 128


def _vq_block(x_ref, xsq_ref, cbt_ref, cb_ref, c2f_ref,
              oh_hbm, xhat_ref, idx_ref, scr, sem):
    G = cb_ref.shape[0]
    CB = cb_ref.shape[1]
    TN = xsq_ref.shape[0]
    i = pl.program_id(0)
    nblk = pl.num_programs(0)
    buf = jax.lax.rem(i, 2)
    iota_c = jax.lax.broadcasted_iota(jnp.int32, (TN, _LC), 1)
    iota_f = jax.lax.broadcasted_iota(jnp.int32, (TN, CB), 1)

    def _copy(b, g):
        return pltpu.make_async_copy(
            scr.at[b, g], oh_hbm.at[pl.ds(i * TN, TN), g, :], sem.at[b, g])

    @pl.when(i >= 2)
    def _wait_prev():
        for g in range(G):
            _copy(buf, g).wait()

    for g in range(G):
        base = g * CB
        prod = jax.lax.dot_general(
            x_ref[:, g, :], cbt_ref[g], (((1,), (0,)), ((), ())),
            preferred_element_type=jnp.float32)
        x2g = xsq_ref[:, g:g + 1]
        rv = (x2g + c2f_ref[0:1, base:base + _LC]) + prod[:, 0:_LC]
        ri = iota_c
        for c in range(1, CB // _LC):
            lo = c * _LC
            d = (x2g + c2f_ref[0:1, base + lo:base + lo + _LC]) + prod[:, lo:lo + _LC]
            upd = d < rv
            ri = jnp.where(upd, iota_c + lo, ri)
            rv = jnp.where(upd, d, rv)
        m = jnp.min(rv, axis=1, keepdims=True)
        cand = jnp.where(rv == m, ri, CB)
        idx = jnp.min(cand, axis=1, keepdims=True)
        oh = (iota_f == idx).astype(jnp.float32)
        scr[buf, g] = oh
        _copy(buf, g).start()
        xhat_ref[:, g, :] = jnp.dot(
            oh, cb_ref[g], preferred_element_type=jnp.float32)
        idx_ref[:, g:g + 1] = idx

    @pl.when(i == nblk - 1)
    def _drain():
        for g in range(G):
            _copy(1 - buf, g).wait()
            _copy(buf, g).wait()


def kernel(x, code_book):
    n, G, dim = x.shape
    CB = code_book.shape[1]
    xsq = jnp.sum(x * x, axis=-1)
    c2f = jnp.sum(code_book * code_book, axis=-1).reshape(1, G * CB)
    cbt = jnp.transpose(-2.0 * code_book, (0, 2, 1))
    one_hot, x_hat, index = pl.pallas_call(
        _vq_block,
        grid=(n // _TN,),
        in_specs=[
            pl.BlockSpec((_TN, G, dim), lambda i: (i, 0, 0)),
            pl.BlockSpec((_TN, G), lambda i: (i, 0)),
            pl.BlockSpec((G, dim, CB), lambda i: (0, 0, 0)),
            pl.BlockSpec((G, CB, dim), lambda i: (0, 0, 0)),
            pl.BlockSpec((1, G * CB), lambda i: (0, 0)),
        ],
        out_specs=[
            pl.BlockSpec(memory_space=pltpu.MemorySpace.HBM),
            pl.BlockSpec((_TN, G, dim), lambda i: (i, 0, 0)),
            pl.BlockSpec((_TN, G), lambda i: (i, 0)),
        ],
        out_shape=[
            jax.ShapeDtypeStruct((n, G, CB), jnp.float32),
            jax.ShapeDtypeStruct((n, G, dim), jnp.float32),
            jax.ShapeDtypeStruct((n, G), jnp.int32),
        ],
        scratch_shapes=[
            pltpu.VMEM((2, 4, _TN, CB), jnp.float32),
            pltpu.SemaphoreType.DMA((2, 4)),
        ],
        compiler_params=pltpu.CompilerParams(
            dimension_semantics=("arbitrary",)),
    )(x, xsq, cbt, code_book, c2f)
    return (x_hat, one_hot, index[..., None])


# speedup vs baseline: 1.2060x; 1.1552x over previous
import jax
import jax.numpy as jnp
from jax.experimental import pallas as pl
from jax.experimental.pallas import tpu as pltpu

_TN = 512
_LC = 128


def _vq_block(x_ref, xsq_ref, cbt_ref, cb_ref, c2f_ref,
              oh_hbm, xhat_ref, idx_ref, scr, sem):
    G = cb_ref.shape[0]
    CB = cb_ref.shape[1]
    TN = xsq_ref.shape[0]
    i = pl.program_id(0)
    nblk = pl.num_programs(0)
    buf = jax.lax.rem(i, 2)
    iota_c = jax.lax.broadcasted_iota(jnp.int32, (TN, _LC), 1)
    iota_f = jax.lax.broadcasted_iota(jnp.int32, (TN, CB), 1)

    def _copy(b, g):
        return pltpu.make_async_copy(
            scr.at[b, g], oh_hbm.at[pl.ds(i * TN, TN), g, :], sem.at[b, g])

    @pl.when(i >= 2)
    def _wait_prev():
        for g in range(G):
            _copy(buf, g).wait()

    for g in range(G):
        base = g * CB
        prod = jax.lax.dot_general(
            x_ref[:, g, :], cbt_ref[g], (((1,), (0,)), ((), ())),
            preferred_element_type=jnp.float32)
        x2g = xsq_ref[:, g:g + 1]
        rv = (x2g + c2f_ref[0:1, base:base + _LC]) + prod[:, 0:_LC]
        ri = iota_c
        for c in range(1, CB // _LC):
            lo = c * _LC
            d = (x2g + c2f_ref[0:1, base + lo:base + lo + _LC]) + prod[:, lo:lo + _LC]
            upd = d < rv
            ri = jnp.where(upd, iota_c + lo, ri)
            rv = jnp.where(upd, d, rv)
        m = jnp.min(rv, axis=1, keepdims=True)
        cand = jnp.where(rv == m, ri, CB)
        idx = jnp.min(cand, axis=1, keepdims=True)
        oh = (iota_f == idx).astype(jnp.float32)
        scr[buf, g] = oh
        _copy(buf, g).start()
        xhat_ref[:, g, :] = jnp.dot(
            oh, cb_ref[g], preferred_element_type=jnp.float32)
        idx_ref[:, g:g + 1] = idx

    @pl.when(i == nblk - 1)
    def _drain():
        for g in range(G):
            _copy(1 - buf, g).wait()
            _copy(buf, g).wait()


def kernel(x, code_book):
    n, G, dim = x.shape
    CB = code_book.shape[1]
    xsq = jnp.sum(x * x, axis=-1)
    c2f = jnp.sum(code_book * code_book, axis=-1).reshape(1, G * CB)
    cbt = jnp.transpose(-2.0 * code_book, (0, 2, 1))
    one_hot, x_hat, index = pl.pallas_call(
        _vq_block,
        grid=(n // _TN,),
        in_specs=[
            pl.BlockSpec((_TN, G, dim), lambda i: (i, 0, 0)),
            pl.BlockSpec((_TN, G), lambda i: (i, 0)),
            pl.BlockSpec((G, dim, CB), lambda i: (0, 0, 0)),
            pl.BlockSpec((G, CB, dim), lambda i: (0, 0, 0)),
            pl.BlockSpec((1, G * CB), lambda i: (0, 0)),
        ],
        out_specs=[
            pl.BlockSpec(memory_space=pltpu.MemorySpace.HBM),
            pl.BlockSpec((_TN, G, dim), lambda i: (i, 0, 0)),
            pl.BlockSpec((_TN, G), lambda i: (i, 0)),
        ],
        out_shape=[
            jax.ShapeDtypeStruct((n, G, CB), jnp.float32),
            jax.ShapeDtypeStruct((n, G, dim), jnp.float32),
            jax.ShapeDtypeStruct((n, G), jnp.int32),
        ],
        scratch_shapes=[
            pltpu.VMEM((2, 4, _TN, CB), jnp.float32),
            pltpu.SemaphoreType.DMA((2, 4)),
        ],
        compiler_params=pltpu.CompilerParams(
            dimension_semantics=("arbitrary",)),
    )(x, xsq, cbt, code_book, c2f)
    return (x_hat, one_hot, index[..., None])


# speedup vs baseline: 1.2406x; 1.0287x over previous
import jax
import jax.numpy as jnp
from jax.experimental import pallas as pl
from jax.experimental.pallas import tpu as pltpu

_TN = 1024
_LC = 128


def _vq_block(x_ref, xsq_ref, cbt_ref, cb_ref, c2f_ref,
              oh_hbm, xhat_ref, idx_ref, scr, sem):
    G = cb_ref.shape[0]
    CB = cb_ref.shape[1]
    TN = xsq_ref.shape[0]
    i = pl.program_id(0)
    nblk = pl.num_programs(0)
    buf = jax.lax.rem(i, 2)
    iota_c = jax.lax.broadcasted_iota(jnp.int32, (TN, _LC), 1)
    iota_f = jax.lax.broadcasted_iota(jnp.int32, (TN, CB), 1)

    def _copy(b, g):
        return pltpu.make_async_copy(
            scr.at[b, g], oh_hbm.at[pl.ds(i * TN, TN), g, :], sem.at[b, g])

    @pl.when(i >= 2)
    def _wait_prev():
        for g in range(G):
            _copy(buf, g).wait()

    for g in range(G):
        base = g * CB
        prod = jax.lax.dot_general(
            x_ref[:, g, :], cbt_ref[g], (((1,), (0,)), ((), ())),
            preferred_element_type=jnp.float32)
        x2g = xsq_ref[:, g:g + 1]
        rv = (x2g + c2f_ref[0:1, base:base + _LC]) + prod[:, 0:_LC]
        ri = iota_c
        for c in range(1, CB // _LC):
            lo = c * _LC
            d = (x2g + c2f_ref[0:1, base + lo:base + lo + _LC]) + prod[:, lo:lo + _LC]
            upd = d < rv
            ri = jnp.where(upd, iota_c + lo, ri)
            rv = jnp.where(upd, d, rv)
        m = jnp.min(rv, axis=1, keepdims=True)
        cand = jnp.where(rv == m, ri, CB)
        idx = jnp.min(cand, axis=1, keepdims=True)
        oh = (iota_f == idx).astype(jnp.float32)
        scr[buf, g] = oh
        _copy(buf, g).start()
        xhat_ref[:, g, :] = jnp.dot(
            oh, cb_ref[g], preferred_element_type=jnp.float32)
        idx_ref[:, g:g + 1] = idx

    @pl.when(i == nblk - 1)
    def _drain():
        for g in range(G):
            _copy(1 - buf, g).wait()
            _copy(buf, g).wait()


def kernel(x, code_book):
    n, G, dim = x.shape
    CB = code_book.shape[1]
    xsq = jnp.sum(x * x, axis=-1)
    c2f = jnp.sum(code_book * code_book, axis=-1).reshape(1, G * CB)
    cbt = jnp.transpose(-2.0 * code_book, (0, 2, 1))
    one_hot, x_hat, index = pl.pallas_call(
        _vq_block,
        grid=(n // _TN,),
        in_specs=[
            pl.BlockSpec((_TN, G, dim), lambda i: (i, 0, 0)),
            pl.BlockSpec((_TN, G), lambda i: (i, 0)),
            pl.BlockSpec((G, dim, CB), lambda i: (0, 0, 0)),
            pl.BlockSpec((G, CB, dim), lambda i: (0, 0, 0)),
            pl.BlockSpec((1, G * CB), lambda i: (0, 0)),
        ],
        out_specs=[
            pl.BlockSpec(memory_space=pltpu.MemorySpace.HBM),
            pl.BlockSpec((_TN, G, dim), lambda i: (i, 0, 0)),
            pl.BlockSpec((_TN, G), lambda i: (i, 0)),
        ],
        out_shape=[
            jax.ShapeDtypeStruct((n, G, CB), jnp.float32),
            jax.ShapeDtypeStruct((n, G, dim), jnp.float32),
            jax.ShapeDtypeStruct((n, G), jnp.int32),
        ],
        scratch_shapes=[
            pltpu.VMEM((2, 4, _TN, CB), jnp.float32),
            pltpu.SemaphoreType.DMA((2, 4)),
        ],
        compiler_params=pltpu.CompilerParams(
            dimension_semantics=("arbitrary",)),
    )(x, xsq, cbt, code_book, c2f)
    return (x_hat, one_hot, index[..., None])
